# baseline (speedup 1.0000x reference)
import functools

import jax
import jax.numpy as jnp
from jax import lax
from jax.experimental import pallas as pl
from jax.experimental.pallas import tpu as pltpu
from jax.experimental.pallas import tpu_sc as plsc

N = 10000
E = 320000
D = 128
DE = 16
H = 4

NC = 2
NS = 16
NW = NC * NS
EW = E // NW
C = 80
NCHUNK = EW // C
RPA = 624
NTAIL = N - NS * RPA
GCH = N // C

_F32 = jnp.float32


@functools.cache
def _mesh():
    return plsc.VectorSubcoreMesh(
        core_axis_name="c", subcore_axis_name="s",
        num_cores=NC, num_subcores=NS)


def _wid():
    return lax.axis_index("s") * NC + lax.axis_index("c")


def _gelu(x):
    return 0.5 * x * (1.0 + lax.erf(x * 0.7071067811865476))


G = 5
NG = NCHUNK // G


@functools.cache
def _gather_rows_kernel():
    @functools.partial(
        pl.kernel,
        out_type=(jax.ShapeDtypeStruct((E, 64), jnp.int32),
                  jax.ShapeDtypeStruct((E, 64), jnp.int32)),
        mesh=_mesh(),
        scratch_types=[pltpu.VMEM((G, C), jnp.int32),
                       pltpu.VMEM((G, C), jnp.int32),
                       pltpu.VMEM((G, C, 64), jnp.int32),
                       pltpu.VMEM((G, C, 64), jnp.int32),
                       pltpu.SemaphoreType.DMA, pltpu.SemaphoreType.DMA,
                       pltpu.SemaphoreType.DMA],
        compiler_params=pltpu.CompilerParams(use_tc_tiling_on_sc=False),
    )
    def _gather_rows(table, idx0, idx1, out0, out1, i0v, i1v, r0v, r1v,
                     si, sg, sw):
        base = _wid() * EW

        def body(g, carry):
            b0 = pl.multiple_of(base + g * (G * C), 8)
            il = []
            for j in range(G):
                bj = pl.multiple_of(b0 + j * C, 8)
                il.append(pltpu.async_copy(idx0.at[pl.ds(bj, C)], i0v.at[j], si))
                il.append(pltpu.async_copy(idx1.at[pl.ds(bj, C)], i1v.at[j], si))

            @pl.when(g > 0)
            def _():
                for j in range(G):
                    pltpu.make_async_copy(r0v.at[j], out0.at[pl.ds(b0, C)], sw).wait()
                    pltpu.make_async_copy(r1v.at[j], out1.at[pl.ds(b0, C)], sw).wait()

            for d in il:
                d.wait()
            gl = []
            for j in range(G):
                gl.append(pltpu.async_copy(table.at[i0v.at[j]], r0v.at[j], sg))
                gl.append(pltpu.async_copy(table.at[i1v.at[j]], r1v.at[j], sg))
            for d in gl:
                d.wait()
            for j in range(G):
                bj = pl.multiple_of(b0 + j * C, 8)
                pltpu.async_copy(r0v.at[j], out0.at[pl.ds(bj, C)], sw)
                pltpu.async_copy(r1v.at[j], out1.at[pl.ds(bj, C)], sw)
            return carry

        lax.fori_loop(0, NG, body, 0)
        for j in range(G):
            pltpu.make_async_copy(r0v.at[j], out0.at[pl.ds(base, C)], sw).wait()
            pltpu.make_async_copy(r1v.at[j], out1.at[pl.ds(base, C)], sw).wait()

    return _gather_rows


_BE = 512


def _edge_fwd_body(src, snk, ea, w1, w2, w3, bv, ablk, ab8,
                   u1, u2, u3, b1, n2w, n2b, n3w, n3b, exp_ref, nnu_ref):
    s = src[...]
    k = snk[...]
    e = ea[...]
    hid = (jnp.dot(s, w1[...], preferred_element_type=_F32)
           + jnp.dot(k, w2[...], preferred_element_type=_F32)
           + jnp.dot(e, w3[...], preferred_element_type=_F32) + bv[...])
    hid = jnp.where(hid >= 0, hid, 0.2 * hid)
    att8 = jnp.dot(hid, ablk[...], preferred_element_type=_F32) + ab8[...]
    exp_ref[...] = jnp.exp(att8)
    h1 = (jnp.dot(s, u1[...], preferred_element_type=_F32)
          + jnp.dot(k, u2[...], preferred_element_type=_F32)
          + jnp.dot(e, u3[...], preferred_element_type=_F32) + b1[...])
    h1 = _gelu(h1)
    h2 = _gelu(jnp.dot(h1, n2w[...], preferred_element_type=_F32) + n2b[...])
    nnu_ref[...] = jnp.dot(h2, n3w[...], preferred_element_type=_F32) + n3b[...]


def _edge_fwd(src_rows, snk_rows, eattr, w1, w2, w3, bv, ablk, ab8,
              u1, u2, u3, b1, n2w, n2b, n3w, n3b):
    grid = (E // _BE,)
    eb = lambda i: (i, 0)
    full = lambda i: (0, 0)
    return pl.pallas_call(
        _edge_fwd_body,
        grid=grid,
        in_specs=[
            pl.BlockSpec((_BE, D), eb), pl.BlockSpec((_BE, D), eb),
            pl.BlockSpec((_BE, DE), eb),
            pl.BlockSpec((D, H * D), full), pl.BlockSpec((D, H * D), full),
            pl.BlockSpec((DE, H * D), full), pl.BlockSpec((1, H * D), full),
            pl.BlockSpec((H * D, 16), full), pl.BlockSpec((1, 16), full),
            pl.BlockSpec((D, D), full), pl.BlockSpec((D, D), full),
            pl.BlockSpec((DE, D), full), pl.BlockSpec((1, D), full),
            pl.BlockSpec((D, D), full), pl.BlockSpec((1, D), full),
            pl.BlockSpec((D, D), full), pl.BlockSpec((1, D), full),
        ],
        out_specs=[pl.BlockSpec((_BE, 16), eb), pl.BlockSpec((_BE, D), eb)],
        out_shape=[jax.ShapeDtypeStruct((E, 16), _F32),
                   jax.ShapeDtypeStruct((E, D), _F32)],
    )(src_rows, snk_rows, eattr, w1, w2, w3, bv, ablk, ab8,
      u1, u2, u3, b1, n2w, n2b, n3w, n3b)


@functools.cache
def _anorm_scatter_kernel():
    @functools.partial(
        pl.kernel,
        out_type=jax.ShapeDtypeStruct((NC, N, 16), _F32),
        mesh=_mesh(),
        scratch_types=[pltpu.VMEM((2, C), jnp.int32),
                       pltpu.VMEM((2, C, 16), _F32),
                       pltpu.VMEM_SHARED((N, 16), _F32),
                       pltpu.SemaphoreType.DMA, pltpu.SemaphoreType.DMA],
        compiler_params=pltpu.CompilerParams(use_tc_tiling_on_sc=False),
    )
    def _anorm_scatter(expatt, seg, zeros8, out, idxv, valv, shared, sl, ss):
        cid = lax.axis_index("c")
        sid = lax.axis_index("s")

        @pl.when(sid == 0)
        def _():
            pltpu.sync_copy(zeros8, shared)

        plsc.subcore_barrier()
        base = _wid() * EW

        def drain(h):
            pltpu.make_async_copy(valv.at[h], shared.at[idxv.at[h]], ss).wait()

        def loads(h, b):
            return [pltpu.async_copy(seg.at[pl.ds(b, C)], idxv.at[h], sl),
                    pltpu.async_copy(expatt.at[pl.ds(b, C)], valv.at[h], sl)]

        def scat(h):
            pltpu.async_copy(valv.at[h], shared.at[idxv.at[h]], ss, add=True)

        def body(m, carry):
            b = pl.multiple_of(base + m * (2 * C), 8)

            @pl.when(m > 0)
            def _():
                drain(0)

            la = loads(0, b)

            @pl.when(m > 0)
            def _():
                drain(1)

            lb = loads(1, pl.multiple_of(b + C, 8))
            for d in la:
                d.wait()
            scat(0)
            for d in lb:
                d.wait()
            scat(1)
            return carry

        lax.fori_loop(0, NCHUNK // 2, body, 0)
        drain(0)
        drain(1)
        bt = pl.multiple_of(base + (NCHUNK - 1) * C, 8)
        for d in loads(0, bt):
            d.wait()
        scat(0)
        drain(0)
        plsc.subcore_barrier()
        r = sid * RPA
        pltpu.sync_copy(shared.at[pl.ds(r, RPA)], out.at[cid, pl.ds(r, RPA)])

        @pl.when(sid == 0)
        def _():
            pltpu.sync_copy(shared.at[pl.ds(NS * RPA, NTAIL)],
                            out.at[cid, pl.ds(NS * RPA, NTAIL)])

    return _anorm_scatter


@functools.cache
def _agg_scatter_kernel():
    @functools.partial(
        pl.kernel,
        out_type=(jax.ShapeDtypeStruct((NC, N, D), _F32),
                  jax.ShapeDtypeStruct((N, 16), _F32)),
        mesh=_mesh(),
        scratch_types=[pltpu.VMEM((2, C), jnp.int32),
                       pltpu.VMEM((2, C, D), _F32),
                       pltpu.VMEM((C,), jnp.int32), pltpu.VMEM((C, 16), _F32),
                       pltpu.SemaphoreType.DMA,
                       pltpu.VMEM_SHARED((N, D), _F32),
                       pltpu.SemaphoreType.DMA, pltpu.SemaphoreType.DMA],
        compiler_params=pltpu.CompilerParams(use_tc_tiling_on_sc=False),
    )
    def _agg_scatter(nnu, seg, segn, anorm, zeros128, out, outg,
                     idxv, valv, gidx, growv, gsem, shared, sl, ss):
        cid = lax.axis_index("c")
        sid = lax.axis_index("s")
        w = _wid()

        @pl.when(sid == 0)
        def _():
            pltpu.sync_copy(zeros128, shared)

        plsc.subcore_barrier()

        def gbody(k, carry):
            ch = k * NW + w

            @pl.when(ch < GCH)
            def _():
                b = pl.multiple_of(ch * C, 8)
                pltpu.sync_copy(segn.at[pl.ds(b, C)], gidx)
                pltpu.async_copy(anorm.at[gidx], growv, gsem).wait()
                pltpu.sync_copy(growv, outg.at[pl.ds(b, C)])

            return carry

        lax.fori_loop(0, (GCH + NW - 1) // NW, gbody, 0)

        base = w * EW

        def drain(h):
            pltpu.make_async_copy(valv.at[h], shared.at[idxv.at[h]], ss).wait()

        def loads(h, b):
            return [pltpu.async_copy(seg.at[pl.ds(b, C)], idxv.at[h], sl),
                    pltpu.async_copy(nnu.at[pl.ds(b, C)], valv.at[h], sl)]

        def scat(h):
            pltpu.async_copy(valv.at[h], shared.at[idxv.at[h]], ss, add=True)

        def body(m, carry):
            b = pl.multiple_of(base + m * (2 * C), 8)

            @pl.when(m > 0)
            def _():
                drain(0)

            la = loads(0, b)

            @pl.when(m > 0)
            def _():
                drain(1)

            lb = loads(1, pl.multiple_of(b + C, 8))
            for d in la:
                d.wait()
            scat(0)
            for d in lb:
                d.wait()
            scat(1)
            return carry

        lax.fori_loop(0, NCHUNK // 2, body, 0)
        drain(0)
        drain(1)
        bt = pl.multiple_of(base + (NCHUNK - 1) * C, 8)
        for d in loads(0, bt):
            d.wait()
        scat(0)
        drain(0)
        plsc.subcore_barrier()
        r = sid * RPA
        pltpu.sync_copy(shared.at[pl.ds(r, RPA)], out.at[cid, pl.ds(r, RPA)])

        @pl.when(sid == 0)
        def _():
            pltpu.sync_copy(shared.at[pl.ds(NS * RPA, NTAIL)],
                            out.at[cid, pl.ds(NS * RPA, NTAIL)])

    return _agg_scatter


_BN = 1000


def _ln(x, g, b):
    mu = jnp.mean(x, axis=-1, keepdims=True)
    var = jnp.mean((x - mu) ** 2, axis=-1, keepdims=True)
    return (x - mu) * lax.rsqrt(var + 1e-5) * g + b


def _node_body(bb, s0, s1, expn, ang, d1w, d1b, d2w, d2b, g1, b1,
               e1s, e1k, e1b, x_ref, ts_ref, tk_ref):
    ratio = expn[...] / jnp.maximum(ang[...], 1e-30)
    coef = 0.25 * jnp.sum(ratio, axis=1, keepdims=True)
    um = coef * (s0[...] + s1[...])
    g = g1[...]
    b = b1[...]
    x1 = _ln(bb[...] + um, g, b)
    hid = _gelu(jnp.dot(x1, d1w[...], preferred_element_type=_F32) + d1b[...])
    dx = jnp.dot(hid, d2w[...], preferred_element_type=_F32) + d2b[...]
    x = _ln(dx + um, g, b)
    x_ref[...] = x
    ts_ref[...] = jnp.dot(x, e1s[...], preferred_element_type=_F32) + e1b[...]
    tk_ref[...] = jnp.dot(x, e1k[...], preferred_element_type=_F32)


def _node_update(bb, s0, s1, expn, ang, d1w, d1b, d2w, d2b, g1, b1,
                 e1s, e1k, e1b):
    grid = (N // _BN,)
    nb = lambda i: (i, 0)
    full = lambda i: (0, 0)
    return pl.pallas_call(
        _node_body,
        grid=grid,
        in_specs=[
            pl.BlockSpec((_BN, D), nb), pl.BlockSpec((_BN, D), nb),
            pl.BlockSpec((_BN, D), nb), pl.BlockSpec((_BN, 16), nb),
            pl.BlockSpec((_BN, 16), nb),
            pl.BlockSpec((D, 4 * D), full), pl.BlockSpec((1, 4 * D), full),
            pl.BlockSpec((4 * D, D), full), pl.BlockSpec((1, D), full),
            pl.BlockSpec((1, D), full), pl.BlockSpec((1, D), full),
            pl.BlockSpec((D, DE), full), pl.BlockSpec((D, DE), full),
            pl.BlockSpec((1, DE), full),
        ],
        out_specs=[pl.BlockSpec((_BN, D), nb), pl.BlockSpec((_BN, DE), nb),
                   pl.BlockSpec((_BN, DE), nb)],
        out_shape=[jax.ShapeDtypeStruct((N, D), _F32),
                   jax.ShapeDtypeStruct((N, DE), _F32),
                   jax.ShapeDtypeStruct((N, DE), _F32)],
    )(bb, s0, s1, expn, ang, d1w, d1b, d2w, d2b, g1, b1, e1s, e1k, e1b)


@functools.cache
def _gather_tables_kernel():
    @functools.partial(
        pl.kernel,
        out_type=(jax.ShapeDtypeStruct((E, DE), _F32),
                  jax.ShapeDtypeStruct((E, DE), _F32)),
        mesh=_mesh(),
        scratch_types=[pltpu.VMEM((G, C), jnp.int32),
                       pltpu.VMEM((G, C), jnp.int32),
                       pltpu.VMEM((G, C, DE), _F32),
                       pltpu.VMEM((G, C, DE), _F32),
                       pltpu.SemaphoreType.DMA, pltpu.SemaphoreType.DMA,
                       pltpu.SemaphoreType.DMA],
        compiler_params=pltpu.CompilerParams(use_tc_tiling_on_sc=False),
    )
    def _gather_tables(ts_tab, tk_tab, idx0, idx1, out0, out1,
                       i0v, i1v, r0v, r1v, si, sg, sw):
        base = _wid() * EW

        def body(g, carry):
            b0 = pl.multiple_of(base + g * (G * C), 8)
            il = []
            for j in range(G):
                bj = pl.multiple_of(b0 + j * C, 8)
                il.append(pltpu.async_copy(idx0.at[pl.ds(bj, C)], i0v.at[j], si))
                il.append(pltpu.async_copy(idx1.at[pl.ds(bj, C)], i1v.at[j], si))

            @pl.when(g > 0)
            def _():
                for j in range(G):
                    pltpu.make_async_copy(r0v.at[j], out0.at[pl.ds(b0, C)], sw).wait()
                    pltpu.make_async_copy(r1v.at[j], out1.at[pl.ds(b0, C)], sw).wait()

            for d in il:
                d.wait()
            gl = []
            for j in range(G):
                gl.append(pltpu.async_copy(ts_tab.at[i0v.at[j]], r0v.at[j], sg))
                gl.append(pltpu.async_copy(tk_tab.at[i1v.at[j]], r1v.at[j], sg))
            for d in gl:
                d.wait()
            for j in range(G):
                bj = pl.multiple_of(b0 + j * C, 8)
                pltpu.async_copy(r0v.at[j], out0.at[pl.ds(bj, C)], sw)
                pltpu.async_copy(r1v.at[j], out1.at[pl.ds(bj, C)], sw)
            return carry

        lax.fori_loop(0, NG, body, 0)
        for j in range(G):
            pltpu.make_async_copy(r0v.at[j], out0.at[pl.ds(base, C)], sw).wait()
            pltpu.make_async_copy(r1v.at[j], out1.at[pl.ds(base, C)], sw).wait()

    return _gather_tables


E8 = E // 8
_BE2 = 2000


def _edge_out_body(ts, tk, ea, e1e, e2w, e2b, e3w, e3b, mg, gg, gb, out_ref):
    e = ea[...]
    t = ts[...] + tk[...] + jnp.dot(e, e1e[...], preferred_element_type=_F32)
    t = _gelu(t)
    t = _gelu(jnp.dot(t, e2w[...], preferred_element_type=_F32) + e2b[...])
    t = jnp.dot(t, e3w[...], preferred_element_type=_F32) + e3b[...]
    y = e + t
    m = mg[...]
    mu = jnp.dot(y, m, preferred_element_type=_F32)
    var = jnp.dot(y * y, m, preferred_element_type=_F32) - mu * mu
    out_ref[...] = (y - mu) * lax.rsqrt(var + 1e-5) * gg[...] + gb[...]


def _edge_out(ts8, tk8, ea8, e1e, e2w, e2b, e3w, e3b, mg, gg, gb):
    grid = (E8 // _BE2,)
    eb = lambda i: (i, 0)
    full = lambda i: (0, 0)
    wspec = pl.BlockSpec((D, D), full)
    bspec = pl.BlockSpec((1, D), full)
    return pl.pallas_call(
        _edge_out_body,
        grid=grid,
        in_specs=[
            pl.BlockSpec((_BE2, D), eb), pl.BlockSpec((_BE2, D), eb),
            pl.BlockSpec((_BE2, D), eb),
            wspec, wspec, bspec, wspec, bspec, wspec, bspec, bspec,
        ],
        out_specs=pl.BlockSpec((_BE2, D), eb),
        out_shape=jax.ShapeDtypeStruct((E8, D), _F32),
    )(ts8, tk8, ea8, e1e, e2w, e2b, e3w, e3b, mg, gg, gb)


def kernel(bb_nodes, eidx, eattr, aW_w, aW_b, aA_w, aA_b, nn1_w, nn1_b,
           nn2_w, nn2_b, nn3_w, nn3_b, dn1_w, dn1_b, dn2_w, dn2_b,
           eu1_w, eu1_b, eu2_w, eu2_b, eu3_w, eu3_b, ln1_g, ln1_b,
           en_g, en_b):
    seg0 = eidx[0]
    seg1 = eidx[1]
    segn = seg1[:N]

    w1 = jnp.transpose(aW_w[:, :D, :], (1, 0, 2)).reshape(D, H * D)
    w2 = jnp.transpose(aW_w[:, D:2 * D, :], (1, 0, 2)).reshape(D, H * D)
    w3 = jnp.transpose(aW_w[:, 2 * D:, :], (1, 0, 2)).reshape(DE, H * D)
    bv = aW_b.reshape(1, H * D)
    eye16 = jax.nn.one_hot(jnp.arange(H), 16, dtype=_F32)
    ablk = (aA_w[:, :, 0][:, :, None] * eye16[:, None, :]).reshape(H * D, 16)
    ab8 = jnp.concatenate(
        [aA_b[:, 0], jnp.full((16 - H,), -1e30, _F32)]).reshape(1, 16)

    u1 = nn1_w[:D]
    u2 = nn1_w[D:2 * D]
    u3 = nn1_w[2 * D:]

    bb16 = bb_nodes.astype(jnp.bfloat16)
    tbl = lax.bitcast_convert_type(bb16.reshape(N, 64, 2), jnp.int32)
    src32, snk32 = _gather_rows_kernel()(tbl, seg0, seg1)
    src_rows = lax.bitcast_convert_type(src32, jnp.bfloat16).reshape(E, D)
    snk_rows = lax.bitcast_convert_type(snk32, jnp.bfloat16).reshape(E, D)
    bf = jnp.bfloat16
    expatt, nnu = _edge_fwd(
        src_rows, snk_rows, eattr, w1.astype(bf), w2.astype(bf), w3, bv,
        ablk, ab8, u1.astype(bf), u2.astype(bf), u3, nn1_b.reshape(1, D),
        nn2_w, nn2_b.reshape(1, D), nn3_w, nn3_b.reshape(1, D))

    zeros8 = jnp.zeros((N, 16), _F32)
    an_parts = _anorm_scatter_kernel()(expatt, seg1, zeros8)
    anorm = an_parts[0] + an_parts[1]

    zeros128 = jnp.zeros((N, D), _F32)
    s_parts, ang = _agg_scatter_kernel()(nnu, seg1, segn, anorm, zeros128)

    x, ts_tab, tk_tab = _node_update(
        bb_nodes, s_parts[0], s_parts[1], expatt[:N], ang,
        dn1_w, dn1_b.reshape(1, 4 * D), dn2_w, dn2_b.reshape(1, D),
        ln1_g.reshape(1, D), ln1_b.reshape(1, D),
        eu1_w[:D], eu1_w[D:2 * D], eu1_b.reshape(1, DE))

    ts_rows, tk_rows = _gather_tables_kernel()(ts_tab, tk_tab, seg0, seg1)
    eye8 = jnp.eye(8, dtype=_F32)
    bd = lambda wmat: jnp.kron(eye8, wmat)
    tile8 = lambda vec: jnp.tile(vec.reshape(1, DE), (1, 8))
    mg = jnp.kron(eye8, jnp.full((DE, DE), 1.0 / DE, _F32))
    ea = _edge_out(
        ts_rows.reshape(E8, D), tk_rows.reshape(E8, D),
        eattr.reshape(E8, D), bd(eu1_w[2 * D:]), bd(eu2_w), tile8(eu2_b),
        bd(eu3_w), tile8(eu3_b), mg, tile8(en_g), tile8(en_b))
    return (x, ea.reshape(E, DE))


# speedup vs baseline: 1.3358x; 1.3358x over previous
import functools

import jax
import jax.numpy as jnp
from jax import lax
from jax.experimental import pallas as pl
from jax.experimental.pallas import tpu as pltpu
from jax.experimental.pallas import tpu_sc as plsc

N = 10000
E = 320000
D = 128
DE = 16
H = 4

NC = 2
NS = 16
NW = NC * NS
EW = E // NW
C = 80
NCHUNK = EW // C
RPA = 624
NTAIL = N - NS * RPA
GCH = N // C

_F32 = jnp.float32


@functools.cache
def _mesh():
    return plsc.VectorSubcoreMesh(
        core_axis_name="c", subcore_axis_name="s",
        num_cores=NC, num_subcores=NS)


def _wid():
    return lax.axis_index("s") * NC + lax.axis_index("c")


def _gelu(x):
    return 0.5 * x * (1.0 + lax.erf(x * 0.7071067811865476))


G = 5
NG = NCHUNK // G


@functools.cache
def _gather_rows_kernel():
    @functools.partial(
        pl.kernel,
        out_type=(jax.ShapeDtypeStruct((E, 64), jnp.int32),
                  jax.ShapeDtypeStruct((E, 64), jnp.int32)),
        mesh=_mesh(),
        scratch_types=[pltpu.VMEM((G, C), jnp.int32),
                       pltpu.VMEM((G, C), jnp.int32),
                       pltpu.VMEM((G, C, 64), jnp.int32),
                       pltpu.VMEM((G, C, 64), jnp.int32),
                       pltpu.SemaphoreType.DMA, pltpu.SemaphoreType.DMA,
                       pltpu.SemaphoreType.DMA],
        compiler_params=pltpu.CompilerParams(use_tc_tiling_on_sc=False),
    )
    def _gather_rows(table, idx0, idx1, out0, out1, i0v, i1v, r0v, r1v,
                     si, sg, sw):
        base = _wid() * EW

        def body(g, carry):
            b0 = pl.multiple_of(base + g * (G * C), 8)
            il = []
            for j in range(G):
                bj = pl.multiple_of(b0 + j * C, 8)
                il.append(pltpu.async_copy(idx0.at[pl.ds(bj, C)], i0v.at[j], si))
                il.append(pltpu.async_copy(idx1.at[pl.ds(bj, C)], i1v.at[j], si))

            @pl.when(g > 0)
            def _():
                for j in range(G):
                    pltpu.make_async_copy(r0v.at[j], out0.at[pl.ds(b0, C)], sw).wait()
                    pltpu.make_async_copy(r1v.at[j], out1.at[pl.ds(b0, C)], sw).wait()

            for d in il:
                d.wait()
            gl = []
            for j in range(G):
                gl.append(pltpu.async_copy(table.at[i0v.at[j]], r0v.at[j], sg))
                gl.append(pltpu.async_copy(table.at[i1v.at[j]], r1v.at[j], sg))
            for d in gl:
                d.wait()
            for j in range(G):
                bj = pl.multiple_of(b0 + j * C, 8)
                pltpu.async_copy(r0v.at[j], out0.at[pl.ds(bj, C)], sw)
                pltpu.async_copy(r1v.at[j], out1.at[pl.ds(bj, C)], sw)
            return carry

        lax.fori_loop(0, NG, body, 0)
        for j in range(G):
            pltpu.make_async_copy(r0v.at[j], out0.at[pl.ds(base, C)], sw).wait()
            pltpu.make_async_copy(r1v.at[j], out1.at[pl.ds(base, C)], sw).wait()

    return _gather_rows


_BE = 512


def _unpack_pair(x32):
    lo = lax.bitcast_convert_type(x32 << 16, _F32)
    hi = lax.bitcast_convert_type(
        x32 & jnp.int32(-65536), _F32)
    return lo, hi


def _edge_fwd_body(src, snk, ea, w1e, w1o, w2e, w2o, w3, bv, ablk, ab8,
                   u1e, u1o, u2e, u2o, u3, b1, n2w, n2b, n3w, n3b,
                   exp_ref, nnu_ref):
    se, so = _unpack_pair(src[...])
    ke, ko = _unpack_pair(snk[...])
    e = ea[...]
    dot = lambda a, b: jnp.dot(a, b, preferred_element_type=_F32)
    hid = (dot(se, w1e[...]) + dot(so, w1o[...])
           + dot(ke, w2e[...]) + dot(ko, w2o[...])
           + dot(e, w3[...]) + bv[...])
    hid = jnp.where(hid >= 0, hid, 0.2 * hid)
    att8 = dot(hid, ablk[...]) + ab8[...]
    exp_ref[...] = jnp.exp(att8)
    h1 = (dot(se, u1e[...]) + dot(so, u1o[...])
          + dot(ke, u2e[...]) + dot(ko, u2o[...])
          + dot(e, u3[...]) + b1[...])
    h1 = _gelu(h1)
    h2 = _gelu(dot(h1, n2w[...]) + n2b[...])
    nnu_ref[...] = dot(h2, n3w[...]) + n3b[...]


def _edge_fwd(src32, snk32, eattr, w1e, w1o, w2e, w2o, w3, bv, ablk, ab8,
              u1e, u1o, u2e, u2o, u3, b1, n2w, n2b, n3w, n3b):
    grid = (E // _BE,)
    eb = lambda i: (i, 0)
    full = lambda i: (0, 0)
    h64 = pl.BlockSpec((64, H * D), full)
    d64 = pl.BlockSpec((64, D), full)
    return pl.pallas_call(
        _edge_fwd_body,
        grid=grid,
        in_specs=[
            pl.BlockSpec((_BE, 64), eb), pl.BlockSpec((_BE, 64), eb),
            pl.BlockSpec((_BE, DE), eb),
            h64, h64, h64, h64,
            pl.BlockSpec((DE, H * D), full), pl.BlockSpec((1, H * D), full),
            pl.BlockSpec((H * D, 16), full), pl.BlockSpec((1, 16), full),
            d64, d64, d64, d64,
            pl.BlockSpec((DE, D), full), pl.BlockSpec((1, D), full),
            pl.BlockSpec((D, D), full), pl.BlockSpec((1, D), full),
            pl.BlockSpec((D, D), full), pl.BlockSpec((1, D), full),
        ],
        out_specs=[pl.BlockSpec((_BE, 16), eb), pl.BlockSpec((_BE, D), eb)],
        out_shape=[jax.ShapeDtypeStruct((E, 16), _F32),
                   jax.ShapeDtypeStruct((E, D), _F32)],
    )(src32, snk32, eattr, w1e, w1o, w2e, w2o, w3, bv, ablk, ab8,
      u1e, u1o, u2e, u2o, u3, b1, n2w, n2b, n3w, n3b)


@functools.cache
def _anorm_scatter_kernel():
    @functools.partial(
        pl.kernel,
        out_type=jax.ShapeDtypeStruct((NC, N, 16), _F32),
        mesh=_mesh(),
        scratch_types=[pltpu.VMEM((2, C), jnp.int32),
                       pltpu.VMEM((2, C, 16), _F32),
                       pltpu.VMEM_SHARED((N, 16), _F32),
                       pltpu.SemaphoreType.DMA, pltpu.SemaphoreType.DMA],
        compiler_params=pltpu.CompilerParams(use_tc_tiling_on_sc=False),
    )
    def _anorm_scatter(expatt, seg, zeros8, out, idxv, valv, shared, sl, ss):
        cid = lax.axis_index("c")
        sid = lax.axis_index("s")

        @pl.when(sid == 0)
        def _():
            pltpu.sync_copy(zeros8, shared)

        plsc.subcore_barrier()
        base = _wid() * EW

        def drain(h):
            pltpu.make_async_copy(valv.at[h], shared.at[idxv.at[h]], ss).wait()

        def loads(h, b):
            return [pltpu.async_copy(seg.at[pl.ds(b, C)], idxv.at[h], sl),
                    pltpu.async_copy(expatt.at[pl.ds(b, C)], valv.at[h], sl)]

        def scat(h):
            pltpu.async_copy(valv.at[h], shared.at[idxv.at[h]], ss, add=True)

        def body(m, carry):
            b = pl.multiple_of(base + m * (2 * C), 8)

            @pl.when(m > 0)
            def _():
                drain(0)

            la = loads(0, b)

            @pl.when(m > 0)
            def _():
                drain(1)

            lb = loads(1, pl.multiple_of(b + C, 8))
            for d in la:
                d.wait()
            scat(0)
            for d in lb:
                d.wait()
            scat(1)
            return carry

        lax.fori_loop(0, NCHUNK // 2, body, 0)
        drain(0)
        drain(1)
        bt = pl.multiple_of(base + (NCHUNK - 1) * C, 8)
        for d in loads(0, bt):
            d.wait()
        scat(0)
        drain(0)
        plsc.subcore_barrier()
        r = sid * RPA
        pltpu.sync_copy(shared.at[pl.ds(r, RPA)], out.at[cid, pl.ds(r, RPA)])

        @pl.when(sid == 0)
        def _():
            pltpu.sync_copy(shared.at[pl.ds(NS * RPA, NTAIL)],
                            out.at[cid, pl.ds(NS * RPA, NTAIL)])

    return _anorm_scatter


@functools.cache
def _agg_scatter_kernel():
    @functools.partial(
        pl.kernel,
        out_type=(jax.ShapeDtypeStruct((NC, N, D), _F32),
                  jax.ShapeDtypeStruct((N, 16), _F32)),
        mesh=_mesh(),
        scratch_types=[pltpu.VMEM((2, C), jnp.int32),
                       pltpu.VMEM((2, C, D), _F32),
                       pltpu.VMEM((C,), jnp.int32), pltpu.VMEM((C, 16), _F32),
                       pltpu.SemaphoreType.DMA,
                       pltpu.VMEM_SHARED((N, D), _F32),
                       pltpu.SemaphoreType.DMA, pltpu.SemaphoreType.DMA],
        compiler_params=pltpu.CompilerParams(use_tc_tiling_on_sc=False),
    )
    def _agg_scatter(nnu, seg, segn, anorm, zeros128, out, outg,
                     idxv, valv, gidx, growv, gsem, shared, sl, ss):
        cid = lax.axis_index("c")
        sid = lax.axis_index("s")
        w = _wid()

        @pl.when(sid == 0)
        def _():
            pltpu.sync_copy(zeros128, shared)

        plsc.subcore_barrier()

        def gbody(k, carry):
            ch = k * NW + w

            @pl.when(ch < GCH)
            def _():
                b = pl.multiple_of(ch * C, 8)
                pltpu.sync_copy(segn.at[pl.ds(b, C)], gidx)
                pltpu.async_copy(anorm.at[gidx], growv, gsem).wait()
                pltpu.sync_copy(growv, outg.at[pl.ds(b, C)])

            return carry

        lax.fori_loop(0, (GCH + NW - 1) // NW, gbody, 0)

        base = w * EW

        def drain(h):
            pltpu.make_async_copy(valv.at[h], shared.at[idxv.at[h]], ss).wait()

        def loads(h, b):
            return [pltpu.async_copy(seg.at[pl.ds(b, C)], idxv.at[h], sl),
                    pltpu.async_copy(nnu.at[pl.ds(b, C)], valv.at[h], sl)]

        def scat(h):
            pltpu.async_copy(valv.at[h], shared.at[idxv.at[h]], ss, add=True)

        def body(m, carry):
            b = pl.multiple_of(base + m * (2 * C), 8)

            @pl.when(m > 0)
            def _():
                drain(0)

            la = loads(0, b)

            @pl.when(m > 0)
            def _():
                drain(1)

            lb = loads(1, pl.multiple_of(b + C, 8))
            for d in la:
                d.wait()
            scat(0)
            for d in lb:
                d.wait()
            scat(1)
            return carry

        lax.fori_loop(0, NCHUNK // 2, body, 0)
        drain(0)
        drain(1)
        bt = pl.multiple_of(base + (NCHUNK - 1) * C, 8)
        for d in loads(0, bt):
            d.wait()
        scat(0)
        drain(0)
        plsc.subcore_barrier()
        r = sid * RPA
        pltpu.sync_copy(shared.at[pl.ds(r, RPA)], out.at[cid, pl.ds(r, RPA)])

        @pl.when(sid == 0)
        def _():
            pltpu.sync_copy(shared.at[pl.ds(NS * RPA, NTAIL)],
                            out.at[cid, pl.ds(NS * RPA, NTAIL)])

    return _agg_scatter


_BN = 1000


def _ln(x, g, b):
    mu = jnp.mean(x, axis=-1, keepdims=True)
    var = jnp.mean((x - mu) ** 2, axis=-1, keepdims=True)
    return (x - mu) * lax.rsqrt(var + 1e-5) * g + b


def _node_body(bb, s0, s1, expn, ang, d1w, d1b, d2w, d2b, g1, b1,
               e1s, e1k, e1b, x_ref, ts_ref, tk_ref):
    ratio = expn[...] / jnp.maximum(ang[...], 1e-30)
    coef = 0.25 * jnp.sum(ratio, axis=1, keepdims=True)
    um = coef * (s0[...] + s1[...])
    g = g1[...]
    b = b1[...]
    x1 = _ln(bb[...] + um, g, b)
    hid = _gelu(jnp.dot(x1, d1w[...], preferred_element_type=_F32) + d1b[...])
    dx = jnp.dot(hid, d2w[...], preferred_element_type=_F32) + d2b[...]
    x = _ln(dx + um, g, b)
    x_ref[...] = x
    ts_ref[...] = jnp.dot(x, e1s[...], preferred_element_type=_F32) + e1b[...]
    tk_ref[...] = jnp.dot(x, e1k[...], preferred_element_type=_F32)


def _node_update(bb, s0, s1, expn, ang, d1w, d1b, d2w, d2b, g1, b1,
                 e1s, e1k, e1b):
    grid = (N // _BN,)
    nb = lambda i: (i, 0)
    full = lambda i: (0, 0)
    return pl.pallas_call(
        _node_body,
        grid=grid,
        in_specs=[
            pl.BlockSpec((_BN, D), nb), pl.BlockSpec((_BN, D), nb),
            pl.BlockSpec((_BN, D), nb), pl.BlockSpec((_BN, 16), nb),
            pl.BlockSpec((_BN, 16), nb),
            pl.BlockSpec((D, 4 * D), full), pl.BlockSpec((1, 4 * D), full),
            pl.BlockSpec((4 * D, D), full), pl.BlockSpec((1, D), full),
            pl.BlockSpec((1, D), full), pl.BlockSpec((1, D), full),
            pl.BlockSpec((D, DE), full), pl.BlockSpec((D, DE), full),
            pl.BlockSpec((1, DE), full),
        ],
        out_specs=[pl.BlockSpec((_BN, D), nb), pl.BlockSpec((_BN, DE), nb),
                   pl.BlockSpec((_BN, DE), nb)],
        out_shape=[jax.ShapeDtypeStruct((N, D), _F32),
                   jax.ShapeDtypeStruct((N, DE), _F32),
                   jax.ShapeDtypeStruct((N, DE), _F32)],
    )(bb, s0, s1, expn, ang, d1w, d1b, d2w, d2b, g1, b1, e1s, e1k, e1b)


@functools.cache
def _gather_tables_kernel():
    @functools.partial(
        pl.kernel,
        out_type=(jax.ShapeDtypeStruct((E, DE), _F32),
                  jax.ShapeDtypeStruct((E, DE), _F32)),
        mesh=_mesh(),
        scratch_types=[pltpu.VMEM((G, C), jnp.int32),
                       pltpu.VMEM((G, C), jnp.int32),
                       pltpu.VMEM((G, C, DE), _F32),
                       pltpu.VMEM((G, C, DE), _F32),
                       pltpu.SemaphoreType.DMA, pltpu.SemaphoreType.DMA,
                       pltpu.SemaphoreType.DMA],
        compiler_params=pltpu.CompilerParams(use_tc_tiling_on_sc=False),
    )
    def _gather_tables(ts_tab, tk_tab, idx0, idx1, out0, out1,
                       i0v, i1v, r0v, r1v, si, sg, sw):
        base = _wid() * EW

        def body(g, carry):
            b0 = pl.multiple_of(base + g * (G * C), 8)
            il = []
            for j in range(G):
                bj = pl.multiple_of(b0 + j * C, 8)
                il.append(pltpu.async_copy(idx0.at[pl.ds(bj, C)], i0v.at[j], si))
                il.append(pltpu.async_copy(idx1.at[pl.ds(bj, C)], i1v.at[j], si))

            @pl.when(g > 0)
            def _():
                for j in range(G):
                    pltpu.make_async_copy(r0v.at[j], out0.at[pl.ds(b0, C)], sw).wait()
                    pltpu.make_async_copy(r1v.at[j], out1.at[pl.ds(b0, C)], sw).wait()

            for d in il:
                d.wait()
            gl = []
            for j in range(G):
                gl.append(pltpu.async_copy(ts_tab.at[i0v.at[j]], r0v.at[j], sg))
                gl.append(pltpu.async_copy(tk_tab.at[i1v.at[j]], r1v.at[j], sg))
            for d in gl:
                d.wait()
            for j in range(G):
                bj = pl.multiple_of(b0 + j * C, 8)
                pltpu.async_copy(r0v.at[j], out0.at[pl.ds(bj, C)], sw)
                pltpu.async_copy(r1v.at[j], out1.at[pl.ds(bj, C)], sw)
            return carry

        lax.fori_loop(0, NG, body, 0)
        for j in range(G):
            pltpu.make_async_copy(r0v.at[j], out0.at[pl.ds(base, C)], sw).wait()
            pltpu.make_async_copy(r1v.at[j], out1.at[pl.ds(base, C)], sw).wait()

    return _gather_tables


_BE2 = 1600


def _ln16(x, g, b):
    mu = jnp.mean(x, axis=-1, keepdims=True)
    var = jnp.mean((x - mu) ** 2, axis=-1, keepdims=True)
    return (x - mu) * lax.rsqrt(var + 1e-5) * g + b


def _edge_out_body(ts, tk, ea, e1e, e2w, e2b, e3w, e3b, gg, gb, out_ref):
    e = ea[...]
    t = ts[...] + tk[...] + jnp.dot(e, e1e[...], preferred_element_type=_F32)
    t = _gelu(t)
    t = _gelu(jnp.dot(t, e2w[...], preferred_element_type=_F32) + e2b[...])
    t = jnp.dot(t, e3w[...], preferred_element_type=_F32) + e3b[...]
    out_ref[...] = _ln16(e + t, gg[...], gb[...])


def _edge_out(ts_rows, tk_rows, eattr, e1e, e2w, e2b, e3w, e3b, gg, gb):
    grid = (E // _BE2,)
    eb = lambda i: (i, 0)
    full = lambda i: (0, 0)
    return pl.pallas_call(
        _edge_out_body,
        grid=grid,
        in_specs=[
            pl.BlockSpec((_BE2, DE), eb), pl.BlockSpec((_BE2, DE), eb),
            pl.BlockSpec((_BE2, DE), eb),
            pl.BlockSpec((DE, DE), full), pl.BlockSpec((DE, DE), full),
            pl.BlockSpec((1, DE), full), pl.BlockSpec((DE, DE), full),
            pl.BlockSpec((1, DE), full), pl.BlockSpec((1, DE), full),
            pl.BlockSpec((1, DE), full),
        ],
        out_specs=pl.BlockSpec((_BE2, DE), eb),
        out_shape=jax.ShapeDtypeStruct((E, DE), _F32),
    )(ts_rows, tk_rows, eattr, e1e, e2w, e2b, e3w, e3b, gg, gb)


def kernel(bb_nodes, eidx, eattr, aW_w, aW_b, aA_w, aA_b, nn1_w, nn1_b,
           nn2_w, nn2_b, nn3_w, nn3_b, dn1_w, dn1_b, dn2_w, dn2_b,
           eu1_w, eu1_b, eu2_w, eu2_b, eu3_w, eu3_b, ln1_g, ln1_b,
           en_g, en_b):
    seg0 = eidx[0]
    seg1 = eidx[1]
    segn = seg1[:N]

    w1 = jnp.transpose(aW_w[:, :D, :], (1, 0, 2)).reshape(D, H * D)
    w2 = jnp.transpose(aW_w[:, D:2 * D, :], (1, 0, 2)).reshape(D, H * D)
    w3 = jnp.transpose(aW_w[:, 2 * D:, :], (1, 0, 2)).reshape(DE, H * D)
    bv = aW_b.reshape(1, H * D)
    eye16 = jax.nn.one_hot(jnp.arange(H), 16, dtype=_F32)
    ablk = (aA_w[:, :, 0][:, :, None] * eye16[:, None, :]).reshape(H * D, 16)
    ab8 = jnp.concatenate(
        [aA_b[:, 0], jnp.full((16 - H,), -1e30, _F32)]).reshape(1, 16)

    u1 = nn1_w[:D]
    u2 = nn1_w[D:2 * D]
    u3 = nn1_w[2 * D:]

    tbl = lax.bitcast_convert_type(
        bb_nodes.astype(jnp.bfloat16).reshape(N, 64, 2), jnp.int32)
    src32, snk32 = _gather_rows_kernel()(tbl, seg0, seg1)
    expatt, nnu = _edge_fwd(
        src32, snk32, eattr, w1[0::2], w1[1::2], w2[0::2], w2[1::2], w3, bv,
        ablk, ab8, u1[0::2], u1[1::2], u2[0::2], u2[1::2], u3,
        nn1_b.reshape(1, D), nn2_w, nn2_b.reshape(1, D),
        nn3_w, nn3_b.reshape(1, D))

    zeros8 = jnp.zeros((N, 16), _F32)
    an_parts = _anorm_scatter_kernel()(expatt, seg1, zeros8)
    anorm = an_parts[0] + an_parts[1]

    zeros128 = jnp.zeros((N, D), _F32)
    s_parts, ang = _agg_scatter_kernel()(nnu, seg1, segn, anorm, zeros128)

    x, ts_tab, tk_tab = _node_update(
        bb_nodes, s_parts[0], s_parts[1], expatt[:N], ang,
        dn1_w, dn1_b.reshape(1, 4 * D), dn2_w, dn2_b.reshape(1, D),
        ln1_g.reshape(1, D), ln1_b.reshape(1, D),
        eu1_w[:D], eu1_w[D:2 * D], eu1_b.reshape(1, DE))

    ts_rows, tk_rows = _gather_tables_kernel()(ts_tab, tk_tab, seg0, seg1)
    ea = _edge_out(ts_rows, tk_rows, eattr, eu1_w[2 * D:], eu2_w,
                   eu2_b.reshape(1, DE), eu3_w, eu3_b.reshape(1, DE),
                   en_g.reshape(1, DE), en_b.reshape(1, DE))
    return (x, ea)


# speedup vs baseline: 1.4732x; 1.1028x over previous
import functools

import jax
import jax.numpy as jnp
from jax import lax
from jax.experimental import pallas as pl
from jax.experimental.pallas import tpu as pltpu
from jax.experimental.pallas import tpu_sc as plsc

N = 10000
E = 320000
D = 128
DE = 16
H = 4

NC = 2
NS = 16
NW = NC * NS
EW = E // NW
C = 80
NCHUNK = EW // C
RPA = 624
NTAIL = N - NS * RPA
GCH = N // C

_F32 = jnp.float32


@functools.cache
def _mesh():
    return plsc.VectorSubcoreMesh(
        core_axis_name="c", subcore_axis_name="s",
        num_cores=NC, num_subcores=NS)


def _wid():
    return lax.axis_index("s") * NC + lax.axis_index("c")


def _gelu(x):
    return 0.5 * x * (1.0 + lax.erf(x * 0.7071067811865476))


G = 5
NG = NCHUNK // G


@functools.cache
def _gather_rows_kernel():
    @functools.partial(
        pl.kernel,
        out_type=(jax.ShapeDtypeStruct((E, 64), jnp.int32),
                  jax.ShapeDtypeStruct((E, 64), jnp.int32)),
        mesh=_mesh(),
        scratch_types=[pltpu.VMEM((G, C), jnp.int32),
                       pltpu.VMEM((G, C), jnp.int32),
                       pltpu.VMEM((G, C, 64), jnp.int32),
                       pltpu.VMEM((G, C, 64), jnp.int32),
                       pltpu.SemaphoreType.DMA, pltpu.SemaphoreType.DMA,
                       pltpu.SemaphoreType.DMA],
        compiler_params=pltpu.CompilerParams(use_tc_tiling_on_sc=False),
    )
    def _gather_rows(table, idx0, idx1, out0, out1, i0v, i1v, r0v, r1v,
                     si, sg, sw):
        base = _wid() * EW

        def body(g, carry):
            b0 = pl.multiple_of(base + g * (G * C), 8)
            il = []
            for j in range(G):
                bj = pl.multiple_of(b0 + j * C, 8)
                il.append(pltpu.async_copy(idx0.at[pl.ds(bj, C)], i0v.at[j], si))
                il.append(pltpu.async_copy(idx1.at[pl.ds(bj, C)], i1v.at[j], si))

            @pl.when(g > 0)
            def _():
                for j in range(G):
                    pltpu.make_async_copy(r0v.at[j], out0.at[pl.ds(b0, C)], sw).wait()
                    pltpu.make_async_copy(r1v.at[j], out1.at[pl.ds(b0, C)], sw).wait()

            for d in il:
                d.wait()
            gl = []
            for j in range(G):
                gl.append(pltpu.async_copy(table.at[i0v.at[j]], r0v.at[j], sg))
                gl.append(pltpu.async_copy(table.at[i1v.at[j]], r1v.at[j], sg))
            for d in gl:
                d.wait()
            for j in range(G):
                bj = pl.multiple_of(b0 + j * C, 8)
                pltpu.async_copy(r0v.at[j], out0.at[pl.ds(bj, C)], sw)
                pltpu.async_copy(r1v.at[j], out1.at[pl.ds(bj, C)], sw)
            return carry

        lax.fori_loop(0, NG, body, 0)
        for j in range(G):
            pltpu.make_async_copy(r0v.at[j], out0.at[pl.ds(base, C)], sw).wait()
            pltpu.make_async_copy(r1v.at[j], out1.at[pl.ds(base, C)], sw).wait()

    return _gather_rows


_BE = 512


def _unpack_pair(x32):
    lo = lax.bitcast_convert_type(x32 << 16, _F32)
    hi = lax.bitcast_convert_type(
        x32 & jnp.int32(-65536), _F32)
    return lo, hi


def _edge_fwd_body(src, snk, ea, wcat, w3, bv, ablk, ab8,
                   ucat, u3, b1, n2w, n2b, n3w, n3b,
                   exp_ref, nnu_ref):
    se, so = _unpack_pair(src[...])
    ke, ko = _unpack_pair(snk[...])
    cat = jnp.concatenate([se, so, ke, ko], axis=1)
    e = ea[...]
    dot = lambda a, b: jnp.dot(a, b, preferred_element_type=_F32)
    hid = dot(cat, wcat[...]) + dot(e, w3[...]) + bv[...]
    hid = jnp.where(hid >= 0, hid, 0.2 * hid)
    att8 = dot(hid, ablk[...]) + ab8[...]
    exp_ref[...] = jnp.exp(att8)
    h1 = dot(cat, ucat[...]) + dot(e, u3[...]) + b1[...]
    h1 = _gelu(h1)
    h2 = _gelu(dot(h1, n2w[...]) + n2b[...])
    nnu_ref[...] = dot(h2, n3w[...]) + n3b[...]


def _edge_fwd(src32, snk32, eattr, wcat, w3, bv, ablk, ab8,
              ucat, u3, b1, n2w, n2b, n3w, n3b):
    grid = (E // _BE,)
    eb = lambda i: (i, 0)
    full = lambda i: (0, 0)
    return pl.pallas_call(
        _edge_fwd_body,
        grid=grid,
        in_specs=[
            pl.BlockSpec((_BE, 64), eb), pl.BlockSpec((_BE, 64), eb),
            pl.BlockSpec((_BE, DE), eb),
            pl.BlockSpec((2 * D, H * D), full),
            pl.BlockSpec((DE, H * D), full), pl.BlockSpec((1, H * D), full),
            pl.BlockSpec((H * D, 16), full), pl.BlockSpec((1, 16), full),
            pl.BlockSpec((2 * D, D), full),
            pl.BlockSpec((DE, D), full), pl.BlockSpec((1, D), full),
            pl.BlockSpec((D, D), full), pl.BlockSpec((1, D), full),
            pl.BlockSpec((D, D), full), pl.BlockSpec((1, D), full),
        ],
        out_specs=[pl.BlockSpec((_BE, 16), eb), pl.BlockSpec((_BE, D), eb)],
        out_shape=[jax.ShapeDtypeStruct((E, 16), _F32),
                   jax.ShapeDtypeStruct((E, D), _F32)],
    )(src32, snk32, eattr, wcat, w3, bv, ablk, ab8,
      ucat, u3, b1, n2w, n2b, n3w, n3b)


@functools.cache
def _anorm_scatter_kernel():
    @functools.partial(
        pl.kernel,
        out_type=jax.ShapeDtypeStruct((NC, N, 16), _F32),
        mesh=_mesh(),
        scratch_types=[pltpu.VMEM((2, C), jnp.int32),
                       pltpu.VMEM((2, C, 16), _F32),
                       pltpu.VMEM_SHARED((N, 16), _F32),
                       pltpu.SemaphoreType.DMA, pltpu.SemaphoreType.DMA],
        compiler_params=pltpu.CompilerParams(use_tc_tiling_on_sc=False),
    )
    def _anorm_scatter(expatt, seg, zeros8, out, idxv, valv, shared, sl, ss):
        cid = lax.axis_index("c")
        sid = lax.axis_index("s")

        @pl.when(sid == 0)
        def _():
            pltpu.sync_copy(zeros8, shared)

        plsc.subcore_barrier()
        base = _wid() * EW

        def drain(h):
            pltpu.make_async_copy(valv.at[h], shared.at[idxv.at[h]], ss).wait()

        def loads(h, b):
            return [pltpu.async_copy(seg.at[pl.ds(b, C)], idxv.at[h], sl),
                    pltpu.async_copy(expatt.at[pl.ds(b, C)], valv.at[h], sl)]

        def scat(h):
            pltpu.async_copy(valv.at[h], shared.at[idxv.at[h]], ss, add=True)

        def body(m, carry):
            b = pl.multiple_of(base + m * (2 * C), 8)

            @pl.when(m > 0)
            def _():
                drain(0)

            la = loads(0, b)

            @pl.when(m > 0)
            def _():
                drain(1)

            lb = loads(1, pl.multiple_of(b + C, 8))
            for d in la:
                d.wait()
            scat(0)
            for d in lb:
                d.wait()
            scat(1)
            return carry

        lax.fori_loop(0, NCHUNK // 2, body, 0)
        drain(0)
        drain(1)
        bt = pl.multiple_of(base + (NCHUNK - 1) * C, 8)
        for d in loads(0, bt):
            d.wait()
        scat(0)
        drain(0)
        plsc.subcore_barrier()
        r = sid * RPA
        pltpu.sync_copy(shared.at[pl.ds(r, RPA)], out.at[cid, pl.ds(r, RPA)])

        @pl.when(sid == 0)
        def _():
            pltpu.sync_copy(shared.at[pl.ds(NS * RPA, NTAIL)],
                            out.at[cid, pl.ds(NS * RPA, NTAIL)])

    return _anorm_scatter


@functools.cache
def _agg_scatter_kernel():
    @functools.partial(
        pl.kernel,
        out_type=(jax.ShapeDtypeStruct((NC, N, D), _F32),
                  jax.ShapeDtypeStruct((N, 16), _F32)),
        mesh=_mesh(),
        scratch_types=[pltpu.VMEM((2, C), jnp.int32),
                       pltpu.VMEM((2, C, D), _F32),
                       pltpu.VMEM((C,), jnp.int32), pltpu.VMEM((C, 16), _F32),
                       pltpu.SemaphoreType.DMA,
                       pltpu.VMEM_SHARED((N, D), _F32),
                       pltpu.SemaphoreType.DMA, pltpu.SemaphoreType.DMA],
        compiler_params=pltpu.CompilerParams(use_tc_tiling_on_sc=False),
    )
    def _agg_scatter(nnu, seg, segn, anorm, zeros128, out, outg,
                     idxv, valv, gidx, growv, gsem, shared, sl, ss):
        cid = lax.axis_index("c")
        sid = lax.axis_index("s")
        w = _wid()

        @pl.when(sid == 0)
        def _():
            pltpu.sync_copy(zeros128, shared)

        plsc.subcore_barrier()

        def gbody(k, carry):
            ch = k * NW + w

            @pl.when(ch < GCH)
            def _():
                b = pl.multiple_of(ch * C, 8)
                pltpu.sync_copy(segn.at[pl.ds(b, C)], gidx)
                pltpu.async_copy(anorm.at[gidx], growv, gsem).wait()
                pltpu.sync_copy(growv, outg.at[pl.ds(b, C)])

            return carry

        lax.fori_loop(0, (GCH + NW - 1) // NW, gbody, 0)

        base = w * EW

        def drain(h):
            pltpu.make_async_copy(valv.at[h], shared.at[idxv.at[h]], ss).wait()

        def loads(h, b):
            return [pltpu.async_copy(seg.at[pl.ds(b, C)], idxv.at[h], sl),
                    pltpu.async_copy(nnu.at[pl.ds(b, C)], valv.at[h], sl)]

        def scat(h):
            pltpu.async_copy(valv.at[h], shared.at[idxv.at[h]], ss, add=True)

        def body(m, carry):
            b = pl.multiple_of(base + m * (2 * C), 8)

            @pl.when(m > 0)
            def _():
                drain(0)

            la = loads(0, b)

            @pl.when(m > 0)
            def _():
                drain(1)

            lb = loads(1, pl.multiple_of(b + C, 8))
            for d in la:
                d.wait()
            scat(0)
            for d in lb:
                d.wait()
            scat(1)
            return carry

        lax.fori_loop(0, NCHUNK // 2, body, 0)
        drain(0)
        drain(1)
        bt = pl.multiple_of(base + (NCHUNK - 1) * C, 8)
        for d in loads(0, bt):
            d.wait()
        scat(0)
        drain(0)
        plsc.subcore_barrier()
        r = sid * RPA
        pltpu.sync_copy(shared.at[pl.ds(r, RPA)], out.at[cid, pl.ds(r, RPA)])

        @pl.when(sid == 0)
        def _():
            pltpu.sync_copy(shared.at[pl.ds(NS * RPA, NTAIL)],
                            out.at[cid, pl.ds(NS * RPA, NTAIL)])

    return _agg_scatter


_BN = 1000


def _ln(x, g, b):
    mu = jnp.mean(x, axis=-1, keepdims=True)
    var = jnp.mean((x - mu) ** 2, axis=-1, keepdims=True)
    return (x - mu) * lax.rsqrt(var + 1e-5) * g + b


def _node_body(bb, s0, s1, expn, ang, d1w, d1b, d2w, d2b, g1, b1,
               e1s, e1k, e1b, x_ref, ts_ref, tk_ref):
    ratio = expn[...] / jnp.maximum(ang[...], 1e-30)
    coef = 0.25 * jnp.sum(ratio, axis=1, keepdims=True)
    um = coef * (s0[...] + s1[...])
    g = g1[...]
    b = b1[...]
    x1 = _ln(bb[...] + um, g, b)
    hid = _gelu(jnp.dot(x1, d1w[...], preferred_element_type=_F32) + d1b[...])
    dx = jnp.dot(hid, d2w[...], preferred_element_type=_F32) + d2b[...]
    x = _ln(dx + um, g, b)
    x_ref[...] = x
    ts_ref[...] = jnp.dot(x, e1s[...], preferred_element_type=_F32) + e1b[...]
    tk_ref[...] = jnp.dot(x, e1k[...], preferred_element_type=_F32)


def _node_update(bb, s0, s1, expn, ang, d1w, d1b, d2w, d2b, g1, b1,
                 e1s, e1k, e1b):
    grid = (N // _BN,)
    nb = lambda i: (i, 0)
    full = lambda i: (0, 0)
    return pl.pallas_call(
        _node_body,
        grid=grid,
        in_specs=[
            pl.BlockSpec((_BN, D), nb), pl.BlockSpec((_BN, D), nb),
            pl.BlockSpec((_BN, D), nb), pl.BlockSpec((_BN, 16), nb),
            pl.BlockSpec((_BN, 16), nb),
            pl.BlockSpec((D, 4 * D), full), pl.BlockSpec((1, 4 * D), full),
            pl.BlockSpec((4 * D, D), full), pl.BlockSpec((1, D), full),
            pl.BlockSpec((1, D), full), pl.BlockSpec((1, D), full),
            pl.BlockSpec((D, DE), full), pl.BlockSpec((D, DE), full),
            pl.BlockSpec((1, DE), full),
        ],
        out_specs=[pl.BlockSpec((_BN, D), nb), pl.BlockSpec((_BN, DE), nb),
                   pl.BlockSpec((_BN, DE), nb)],
        out_shape=[jax.ShapeDtypeStruct((N, D), _F32),
                   jax.ShapeDtypeStruct((N, DE), _F32),
                   jax.ShapeDtypeStruct((N, DE), _F32)],
    )(bb, s0, s1, expn, ang, d1w, d1b, d2w, d2b, g1, b1, e1s, e1k, e1b)


@functools.cache
def _gather_tables_kernel():
    @functools.partial(
        pl.kernel,
        out_type=(jax.ShapeDtypeStruct((E, DE), _F32),
                  jax.ShapeDtypeStruct((E, DE), _F32)),
        mesh=_mesh(),
        scratch_types=[pltpu.VMEM((G, C), jnp.int32),
                       pltpu.VMEM((G, C), jnp.int32),
                       pltpu.VMEM((G, C, DE), _F32),
                       pltpu.VMEM((G, C, DE), _F32),
                       pltpu.SemaphoreType.DMA, pltpu.SemaphoreType.DMA,
                       pltpu.SemaphoreType.DMA],
        compiler_params=pltpu.CompilerParams(use_tc_tiling_on_sc=False),
    )
    def _gather_tables(ts_tab, tk_tab, idx0, idx1, out0, out1,
                       i0v, i1v, r0v, r1v, si, sg, sw):
        base = _wid() * EW

        def body(g, carry):
            b0 = pl.multiple_of(base + g * (G * C), 8)
            il = []
            for j in range(G):
                bj = pl.multiple_of(b0 + j * C, 8)
                il.append(pltpu.async_copy(idx0.at[pl.ds(bj, C)], i0v.at[j], si))
                il.append(pltpu.async_copy(idx1.at[pl.ds(bj, C)], i1v.at[j], si))

            @pl.when(g > 0)
            def _():
                for j in range(G):
                    pltpu.make_async_copy(r0v.at[j], out0.at[pl.ds(b0, C)], sw).wait()
                    pltpu.make_async_copy(r1v.at[j], out1.at[pl.ds(b0, C)], sw).wait()

            for d in il:
                d.wait()
            gl = []
            for j in range(G):
                gl.append(pltpu.async_copy(ts_tab.at[i0v.at[j]], r0v.at[j], sg))
                gl.append(pltpu.async_copy(tk_tab.at[i1v.at[j]], r1v.at[j], sg))
            for d in gl:
                d.wait()
            for j in range(G):
                bj = pl.multiple_of(b0 + j * C, 8)
                pltpu.async_copy(r0v.at[j], out0.at[pl.ds(bj, C)], sw)
                pltpu.async_copy(r1v.at[j], out1.at[pl.ds(bj, C)], sw)
            return carry

        lax.fori_loop(0, NG, body, 0)
        for j in range(G):
            pltpu.make_async_copy(r0v.at[j], out0.at[pl.ds(base, C)], sw).wait()
            pltpu.make_async_copy(r1v.at[j], out1.at[pl.ds(base, C)], sw).wait()

    return _gather_tables


_BE2 = 1600


def _ln16(x, g, b):
    mu = jnp.mean(x, axis=-1, keepdims=True)
    var = jnp.mean((x - mu) ** 2, axis=-1, keepdims=True)
    return (x - mu) * lax.rsqrt(var + 1e-5) * g + b


def _edge_out_body(ts, tk, ea, e1e, e2w, e2b, e3w, e3b, gg, gb, out_ref):
    e = ea[...]
    t = ts[...] + tk[...] + jnp.dot(e, e1e[...], preferred_element_type=_F32)
    t = _gelu(t)
    t = _gelu(jnp.dot(t, e2w[...], preferred_element_type=_F32) + e2b[...])
    t = jnp.dot(t, e3w[...], preferred_element_type=_F32) + e3b[...]
    out_ref[...] = _ln16(e + t, gg[...], gb[...])


def _edge_out(ts_rows, tk_rows, eattr, e1e, e2w, e2b, e3w, e3b, gg, gb):
    grid = (E // _BE2,)
    eb = lambda i: (i, 0)
    full = lambda i: (0, 0)
    return pl.pallas_call(
        _edge_out_body,
        grid=grid,
        in_specs=[
            pl.BlockSpec((_BE2, DE), eb), pl.BlockSpec((_BE2, DE), eb),
            pl.BlockSpec((_BE2, DE), eb),
            pl.BlockSpec((DE, DE), full), pl.BlockSpec((DE, DE), full),
            pl.BlockSpec((1, DE), full), pl.BlockSpec((DE, DE), full),
            pl.BlockSpec((1, DE), full), pl.BlockSpec((1, DE), full),
            pl.BlockSpec((1, DE), full),
        ],
        out_specs=pl.BlockSpec((_BE2, DE), eb),
        out_shape=jax.ShapeDtypeStruct((E, DE), _F32),
    )(ts_rows, tk_rows, eattr, e1e, e2w, e2b, e3w, e3b, gg, gb)


def kernel(bb_nodes, eidx, eattr, aW_w, aW_b, aA_w, aA_b, nn1_w, nn1_b,
           nn2_w, nn2_b, nn3_w, nn3_b, dn1_w, dn1_b, dn2_w, dn2_b,
           eu1_w, eu1_b, eu2_w, eu2_b, eu3_w, eu3_b, ln1_g, ln1_b,
           en_g, en_b):
    seg0 = eidx[0]
    seg1 = eidx[1]
    segn = seg1[:N]

    w1 = jnp.transpose(aW_w[:, :D, :], (1, 0, 2)).reshape(D, H * D)
    w2 = jnp.transpose(aW_w[:, D:2 * D, :], (1, 0, 2)).reshape(D, H * D)
    w3 = jnp.transpose(aW_w[:, 2 * D:, :], (1, 0, 2)).reshape(DE, H * D)
    bv = aW_b.reshape(1, H * D)
    eye16 = jax.nn.one_hot(jnp.arange(H), 16, dtype=_F32)
    ablk = (aA_w[:, :, 0][:, :, None] * eye16[:, None, :]).reshape(H * D, 16)
    ab8 = jnp.concatenate(
        [aA_b[:, 0], jnp.full((16 - H,), -1e30, _F32)]).reshape(1, 16)

    u1 = nn1_w[:D]
    u2 = nn1_w[D:2 * D]
    u3 = nn1_w[2 * D:]

    tbl = lax.bitcast_convert_type(
        bb_nodes.astype(jnp.bfloat16).reshape(N, 64, 2), jnp.int32)
    src32, snk32 = _gather_rows_kernel()(tbl, seg0, seg1)
    wcat = jnp.concatenate([w1[0::2], w1[1::2], w2[0::2], w2[1::2]], axis=0)
    ucat = jnp.concatenate([u1[0::2], u1[1::2], u2[0::2], u2[1::2]], axis=0)
    expatt, nnu = _edge_fwd(
        src32, snk32, eattr, wcat, w3, bv, ablk, ab8, ucat, u3,
        nn1_b.reshape(1, D), nn2_w, nn2_b.reshape(1, D),
        nn3_w, nn3_b.reshape(1, D))

    zeros8 = jnp.zeros((N, 16), _F32)
    an_parts = _anorm_scatter_kernel()(expatt, seg1, zeros8)
    anorm = an_parts[0] + an_parts[1]

    zeros128 = jnp.zeros((N, D), _F32)
    s_parts, ang = _agg_scatter_kernel()(nnu, seg1, segn, anorm, zeros128)

    x, ts_tab, tk_tab = _node_update(
        bb_nodes, s_parts[0], s_parts[1], expatt[:N], ang,
        dn1_w, dn1_b.reshape(1, 4 * D), dn2_w, dn2_b.reshape(1, D),
        ln1_g.reshape(1, D), ln1_b.reshape(1, D),
        eu1_w[:D], eu1_w[D:2 * D], eu1_b.reshape(1, DE))

    ts_rows, tk_rows = _gather_tables_kernel()(ts_tab, tk_tab, seg0, seg1)
    ea = _edge_out(ts_rows, tk_rows, eattr, eu1_w[2 * D:], eu2_w,
                   eu2_b.reshape(1, DE), eu3_w, eu3_b.reshape(1, DE),
                   en_g.reshape(1, DE), en_b.reshape(1, DE))
    return (x, ea)


# speedup vs baseline: 1.6806x; 1.1408x over previous
import functools

import jax
import jax.numpy as jnp
from jax import lax
from jax.experimental import pallas as pl
from jax.experimental.pallas import tpu as pltpu
from jax.experimental.pallas import tpu_sc as plsc

N = 10000
E = 320000
D = 128
DE = 16
H = 4

NC = 2
NS = 16
NW = NC * NS
EW = E // NW
C = 80
NCHUNK = EW // C
RPA = 624
NTAIL = N - NS * RPA
GCH = N // C

_F32 = jnp.float32


@functools.cache
def _mesh():
    return plsc.VectorSubcoreMesh(
        core_axis_name="c", subcore_axis_name="s",
        num_cores=NC, num_subcores=NS)


def _wid():
    return lax.axis_index("s") * NC + lax.axis_index("c")


def _gelu(x):
    return 0.5 * x * (1.0 + lax.erf(x * 0.7071067811865476))


G = 5
NG = NCHUNK // G


@functools.cache
def _gather_rows_kernel():
    @functools.partial(
        pl.kernel,
        out_type=(jax.ShapeDtypeStruct((E, 64), jnp.int32),
                  jax.ShapeDtypeStruct((E, 64), jnp.int32)),
        mesh=_mesh(),
        scratch_types=[pltpu.VMEM((G, C), jnp.int32),
                       pltpu.VMEM((G, C), jnp.int32),
                       pltpu.VMEM((G, C, 64), jnp.int32),
                       pltpu.VMEM((G, C, 64), jnp.int32),
                       pltpu.SemaphoreType.DMA, pltpu.SemaphoreType.DMA,
                       pltpu.SemaphoreType.DMA],
        compiler_params=pltpu.CompilerParams(use_tc_tiling_on_sc=False),
    )
    def _gather_rows(table, idx0, idx1, out0, out1, i0v, i1v, r0v, r1v,
                     si, sg, sw):
        base = _wid() * EW

        def body(g, carry):
            b0 = pl.multiple_of(base + g * (G * C), 8)
            il = []
            for j in range(G):
                bj = pl.multiple_of(b0 + j * C, 8)
                il.append(pltpu.async_copy(idx0.at[pl.ds(bj, C)], i0v.at[j], si))
                il.append(pltpu.async_copy(idx1.at[pl.ds(bj, C)], i1v.at[j], si))

            @pl.when(g > 0)
            def _():
                for j in range(G):
                    pltpu.make_async_copy(r0v.at[j], out0.at[pl.ds(b0, C)], sw).wait()
                    pltpu.make_async_copy(r1v.at[j], out1.at[pl.ds(b0, C)], sw).wait()

            for d in il:
                d.wait()
            gl = []
            for j in range(G):
                gl.append(pltpu.async_copy(table.at[i0v.at[j]], r0v.at[j], sg))
                gl.append(pltpu.async_copy(table.at[i1v.at[j]], r1v.at[j], sg))
            for d in gl:
                d.wait()
            for j in range(G):
                bj = pl.multiple_of(b0 + j * C, 8)
                pltpu.async_copy(r0v.at[j], out0.at[pl.ds(bj, C)], sw)
                pltpu.async_copy(r1v.at[j], out1.at[pl.ds(bj, C)], sw)
            return carry

        lax.fori_loop(0, NG, body, 0)
        for j in range(G):
            pltpu.make_async_copy(r0v.at[j], out0.at[pl.ds(base, C)], sw).wait()
            pltpu.make_async_copy(r1v.at[j], out1.at[pl.ds(base, C)], sw).wait()

    return _gather_rows


_BE = 1024


def _unpack_pair(x32):
    lo = lax.bitcast_convert_type(x32 << 16, _F32)
    hi = lax.bitcast_convert_type(
        x32 & jnp.int32(-65536), _F32)
    return lo, hi


def _edge_fwd_body(src, snk, ea, wcat, w3, bv, ablk, ab8,
                   ucat, u3, b1, n2w, n2b, n3w, n3b,
                   exp_ref, nnu_ref):
    se, so = _unpack_pair(src[...])
    ke, ko = _unpack_pair(snk[...])
    cat = jnp.concatenate([se, so, ke, ko], axis=1)
    e = ea[...]
    dot = lambda a, b: jnp.dot(a, b, preferred_element_type=_F32)
    hid = dot(cat, wcat[...]) + dot(e, w3[...]) + bv[...]
    hid = jnp.where(hid >= 0, hid, 0.2 * hid)
    att8 = dot(hid, ablk[...]) + ab8[...]
    exp_ref[...] = jnp.exp(att8)
    h1 = dot(cat, ucat[...]) + dot(e, u3[...]) + b1[...]
    h1 = _gelu(h1)
    h2 = _gelu(dot(h1, n2w[...]) + n2b[...])
    nnu_ref[...] = dot(h2, n3w[...]) + n3b[...]


def _edge_fwd(src32, snk32, eattr, wcat, w3, bv, ablk, ab8,
              ucat, u3, b1, n2w, n2b, n3w, n3b):
    grid = (E // _BE,)
    eb = lambda i: (i, 0)
    full = lambda i: (0, 0)
    return pl.pallas_call(
        _edge_fwd_body,
        grid=grid,
        in_specs=[
            pl.BlockSpec((_BE, 64), eb), pl.BlockSpec((_BE, 64), eb),
            pl.BlockSpec((_BE, DE), eb),
            pl.BlockSpec((2 * D, H * D), full),
            pl.BlockSpec((DE, H * D), full), pl.BlockSpec((1, H * D), full),
            pl.BlockSpec((H * D, 16), full), pl.BlockSpec((1, 16), full),
            pl.BlockSpec((2 * D, D), full),
            pl.BlockSpec((DE, D), full), pl.BlockSpec((1, D), full),
            pl.BlockSpec((D, D), full), pl.BlockSpec((1, D), full),
            pl.BlockSpec((D, D), full), pl.BlockSpec((1, D), full),
        ],
        out_specs=[pl.BlockSpec((_BE, 16), eb), pl.BlockSpec((_BE, D), eb)],
        out_shape=[jax.ShapeDtypeStruct((E, 16), _F32),
                   jax.ShapeDtypeStruct((E, D), _F32)],
    )(src32, snk32, eattr, wcat, w3, bv, ablk, ab8,
      ucat, u3, b1, n2w, n2b, n3w, n3b)


@functools.cache
def _anorm_scatter_kernel():
    @functools.partial(
        pl.kernel,
        out_type=jax.ShapeDtypeStruct((NC, N, 16), _F32),
        mesh=_mesh(),
        scratch_types=[pltpu.VMEM((2, C), jnp.int32),
                       pltpu.VMEM((2, C, 16), _F32),
                       pltpu.VMEM_SHARED((N, 16), _F32),
                       pltpu.SemaphoreType.DMA, pltpu.SemaphoreType.DMA],
        compiler_params=pltpu.CompilerParams(use_tc_tiling_on_sc=False),
    )
    def _anorm_scatter(expatt, seg, zeros8, out, idxv, valv, shared, sl, ss):
        cid = lax.axis_index("c")
        sid = lax.axis_index("s")

        @pl.when(sid == 0)
        def _():
            pltpu.sync_copy(zeros8, shared)

        plsc.subcore_barrier()
        base = _wid() * EW

        def drain(h):
            pltpu.make_async_copy(valv.at[h], shared.at[idxv.at[h]], ss).wait()

        def loads(h, b):
            return [pltpu.async_copy(seg.at[pl.ds(b, C)], idxv.at[h], sl),
                    pltpu.async_copy(expatt.at[pl.ds(b, C)], valv.at[h], sl)]

        def scat(h):
            pltpu.async_copy(valv.at[h], shared.at[idxv.at[h]], ss, add=True)

        def body(m, carry):
            b = pl.multiple_of(base + m * (2 * C), 8)

            @pl.when(m > 0)
            def _():
                drain(0)

            la = loads(0, b)

            @pl.when(m > 0)
            def _():
                drain(1)

            lb = loads(1, pl.multiple_of(b + C, 8))
            for d in la:
                d.wait()
            scat(0)
            for d in lb:
                d.wait()
            scat(1)
            return carry

        lax.fori_loop(0, NCHUNK // 2, body, 0)
        drain(0)
        drain(1)
        bt = pl.multiple_of(base + (NCHUNK - 1) * C, 8)
        for d in loads(0, bt):
            d.wait()
        scat(0)
        drain(0)
        plsc.subcore_barrier()
        r = sid * RPA
        pltpu.sync_copy(shared.at[pl.ds(r, RPA)], out.at[cid, pl.ds(r, RPA)])

        @pl.when(sid == 0)
        def _():
            pltpu.sync_copy(shared.at[pl.ds(NS * RPA, NTAIL)],
                            out.at[cid, pl.ds(NS * RPA, NTAIL)])

    return _anorm_scatter


@functools.cache
def _agg_scatter_kernel():
    @functools.partial(
        pl.kernel,
        out_type=(jax.ShapeDtypeStruct((NC, N, D), _F32),
                  jax.ShapeDtypeStruct((N, 16), _F32)),
        mesh=_mesh(),
        scratch_types=[pltpu.VMEM((2, C), jnp.int32),
                       pltpu.VMEM((2, C, D), _F32),
                       pltpu.VMEM((C,), jnp.int32), pltpu.VMEM((C, 16), _F32),
                       pltpu.SemaphoreType.DMA,
                       pltpu.VMEM_SHARED((N, D), _F32),
                       pltpu.SemaphoreType.DMA, pltpu.SemaphoreType.DMA],
        compiler_params=pltpu.CompilerParams(use_tc_tiling_on_sc=False),
    )
    def _agg_scatter(nnu, seg, segn, anorm, zeros128, out, outg,
                     idxv, valv, gidx, growv, gsem, shared, sl, ss):
        cid = lax.axis_index("c")
        sid = lax.axis_index("s")
        w = _wid()

        @pl.when(sid == 0)
        def _():
            pltpu.sync_copy(zeros128, shared)

        plsc.subcore_barrier()

        def gbody(k, carry):
            ch = k * NW + w

            @pl.when(ch < GCH)
            def _():
                b = pl.multiple_of(ch * C, 8)
                pltpu.sync_copy(segn.at[pl.ds(b, C)], gidx)
                pltpu.async_copy(anorm.at[gidx], growv, gsem).wait()
                pltpu.sync_copy(growv, outg.at[pl.ds(b, C)])

            return carry

        lax.fori_loop(0, (GCH + NW - 1) // NW, gbody, 0)

        base = w * EW

        def drain(h):
            pltpu.make_async_copy(valv.at[h], shared.at[idxv.at[h]], ss).wait()

        def loads(h, b):
            return [pltpu.async_copy(seg.at[pl.ds(b, C)], idxv.at[h], sl),
                    pltpu.async_copy(nnu.at[pl.ds(b, C)], valv.at[h], sl)]

        def scat(h):
            pltpu.async_copy(valv.at[h], shared.at[idxv.at[h]], ss, add=True)

        def body(m, carry):
            b = pl.multiple_of(base + m * (2 * C), 8)

            @pl.when(m > 0)
            def _():
                drain(0)

            la = loads(0, b)

            @pl.when(m > 0)
            def _():
                drain(1)

            lb = loads(1, pl.multiple_of(b + C, 8))
            for d in la:
                d.wait()
            scat(0)
            for d in lb:
                d.wait()
            scat(1)
            return carry

        lax.fori_loop(0, NCHUNK // 2, body, 0)
        drain(0)
        drain(1)
        bt = pl.multiple_of(base + (NCHUNK - 1) * C, 8)
        for d in loads(0, bt):
            d.wait()
        scat(0)
        drain(0)
        plsc.subcore_barrier()
        r = sid * RPA
        pltpu.sync_copy(shared.at[pl.ds(r, RPA)], out.at[cid, pl.ds(r, RPA)])

        @pl.when(sid == 0)
        def _():
            pltpu.sync_copy(shared.at[pl.ds(NS * RPA, NTAIL)],
                            out.at[cid, pl.ds(NS * RPA, NTAIL)])

    return _agg_scatter


_BN = 1000


def _ln(x, g, b):
    mu = jnp.mean(x, axis=-1, keepdims=True)
    var = jnp.mean((x - mu) ** 2, axis=-1, keepdims=True)
    return (x - mu) * lax.rsqrt(var + 1e-5) * g + b


def _node_body(bb, s0, s1, expn, ang, d1w, d1b, d2w, d2b, g1, b1,
               e1s, e1k, e1b, x_ref, ts_ref, tk_ref):
    ratio = expn[...] / jnp.maximum(ang[...], 1e-30)
    coef = 0.25 * jnp.sum(ratio, axis=1, keepdims=True)
    um = coef * (s0[...] + s1[...])
    g = g1[...]
    b = b1[...]
    x1 = _ln(bb[...] + um, g, b)
    hid = _gelu(jnp.dot(x1, d1w[...], preferred_element_type=_F32) + d1b[...])
    dx = jnp.dot(hid, d2w[...], preferred_element_type=_F32) + d2b[...]
    x = _ln(dx + um, g, b)
    x_ref[...] = x
    ts_ref[...] = jnp.dot(x, e1s[...], preferred_element_type=_F32) + e1b[...]
    tk_ref[...] = jnp.dot(x, e1k[...], preferred_element_type=_F32)


def _node_update(bb, s0, s1, expn, ang, d1w, d1b, d2w, d2b, g1, b1,
                 e1s, e1k, e1b):
    grid = (N // _BN,)
    nb = lambda i: (i, 0)
    full = lambda i: (0, 0)
    return pl.pallas_call(
        _node_body,
        grid=grid,
        in_specs=[
            pl.BlockSpec((_BN, D), nb), pl.BlockSpec((_BN, D), nb),
            pl.BlockSpec((_BN, D), nb), pl.BlockSpec((_BN, 16), nb),
            pl.BlockSpec((_BN, 16), nb),
            pl.BlockSpec((D, 4 * D), full), pl.BlockSpec((1, 4 * D), full),
            pl.BlockSpec((4 * D, D), full), pl.BlockSpec((1, D), full),
            pl.BlockSpec((1, D), full), pl.BlockSpec((1, D), full),
            pl.BlockSpec((D, DE), full), pl.BlockSpec((D, DE), full),
            pl.BlockSpec((1, DE), full),
        ],
        out_specs=[pl.BlockSpec((_BN, D), nb), pl.BlockSpec((_BN, DE), nb),
                   pl.BlockSpec((_BN, DE), nb)],
        out_shape=[jax.ShapeDtypeStruct((N, D), _F32),
                   jax.ShapeDtypeStruct((N, DE), _F32),
                   jax.ShapeDtypeStruct((N, DE), _F32)],
    )(bb, s0, s1, expn, ang, d1w, d1b, d2w, d2b, g1, b1, e1s, e1k, e1b)


@functools.cache
def _gather_tables_kernel():
    @functools.partial(
        pl.kernel,
        out_type=(jax.ShapeDtypeStruct((E, DE), _F32),
                  jax.ShapeDtypeStruct((E, DE), _F32)),
        mesh=_mesh(),
        scratch_types=[pltpu.VMEM((G, C), jnp.int32),
                       pltpu.VMEM((G, C), jnp.int32),
                       pltpu.VMEM((G, C, DE), _F32),
                       pltpu.VMEM((G, C, DE), _F32),
                       pltpu.SemaphoreType.DMA, pltpu.SemaphoreType.DMA,
                       pltpu.SemaphoreType.DMA],
        compiler_params=pltpu.CompilerParams(use_tc_tiling_on_sc=False),
    )
    def _gather_tables(ts_tab, tk_tab, idx0, idx1, out0, out1,
                       i0v, i1v, r0v, r1v, si, sg, sw):
        base = _wid() * EW

        def body(g, carry):
            b0 = pl.multiple_of(base + g * (G * C), 8)
            il = []
            for j in range(G):
                bj = pl.multiple_of(b0 + j * C, 8)
                il.append(pltpu.async_copy(idx0.at[pl.ds(bj, C)], i0v.at[j], si))
                il.append(pltpu.async_copy(idx1.at[pl.ds(bj, C)], i1v.at[j], si))

            @pl.when(g > 0)
            def _():
                for j in range(G):
                    pltpu.make_async_copy(r0v.at[j], out0.at[pl.ds(b0, C)], sw).wait()
                    pltpu.make_async_copy(r1v.at[j], out1.at[pl.ds(b0, C)], sw).wait()

            for d in il:
                d.wait()
            gl = []
            for j in range(G):
                gl.append(pltpu.async_copy(ts_tab.at[i0v.at[j]], r0v.at[j], sg))
                gl.append(pltpu.async_copy(tk_tab.at[i1v.at[j]], r1v.at[j], sg))
            for d in gl:
                d.wait()
            for j in range(G):
                bj = pl.multiple_of(b0 + j * C, 8)
                pltpu.async_copy(r0v.at[j], out0.at[pl.ds(bj, C)], sw)
                pltpu.async_copy(r1v.at[j], out1.at[pl.ds(bj, C)], sw)
            return carry

        lax.fori_loop(0, NG, body, 0)
        for j in range(G):
            pltpu.make_async_copy(r0v.at[j], out0.at[pl.ds(base, C)], sw).wait()
            pltpu.make_async_copy(r1v.at[j], out1.at[pl.ds(base, C)], sw).wait()

    return _gather_tables


_BE2 = 3200


def _ln16(x, g, b):
    mu = jnp.mean(x, axis=-1, keepdims=True)
    var = jnp.mean((x - mu) ** 2, axis=-1, keepdims=True)
    return (x - mu) * lax.rsqrt(var + 1e-5) * g + b


def _edge_out_body(ts, tk, ea, e1e, e2w, e2b, e3w, e3b, gg, gb, out_ref):
    e = ea[...]
    t = ts[...] + tk[...] + jnp.dot(e, e1e[...], preferred_element_type=_F32)
    t = _gelu(t)
    t = _gelu(jnp.dot(t, e2w[...], preferred_element_type=_F32) + e2b[...])
    t = jnp.dot(t, e3w[...], preferred_element_type=_F32) + e3b[...]
    out_ref[...] = _ln16(e + t, gg[...], gb[...])


def _edge_out(ts_rows, tk_rows, eattr, e1e, e2w, e2b, e3w, e3b, gg, gb):
    grid = (E // _BE2,)
    eb = lambda i: (i, 0)
    full = lambda i: (0, 0)
    return pl.pallas_call(
        _edge_out_body,
        grid=grid,
        in_specs=[
            pl.BlockSpec((_BE2, DE), eb), pl.BlockSpec((_BE2, DE), eb),
            pl.BlockSpec((_BE2, DE), eb),
            pl.BlockSpec((DE, DE), full), pl.BlockSpec((DE, DE), full),
            pl.BlockSpec((1, DE), full), pl.BlockSpec((DE, DE), full),
            pl.BlockSpec((1, DE), full), pl.BlockSpec((1, DE), full),
            pl.BlockSpec((1, DE), full),
        ],
        out_specs=pl.BlockSpec((_BE2, DE), eb),
        out_shape=jax.ShapeDtypeStruct((E, DE), _F32),
    )(ts_rows, tk_rows, eattr, e1e, e2w, e2b, e3w, e3b, gg, gb)


def kernel(bb_nodes, eidx, eattr, aW_w, aW_b, aA_w, aA_b, nn1_w, nn1_b,
           nn2_w, nn2_b, nn3_w, nn3_b, dn1_w, dn1_b, dn2_w, dn2_b,
           eu1_w, eu1_b, eu2_w, eu2_b, eu3_w, eu3_b, ln1_g, ln1_b,
           en_g, en_b):
    seg0 = eidx[0]
    seg1 = eidx[1]
    segn = seg1[:N]

    w1 = jnp.transpose(aW_w[:, :D, :], (1, 0, 2)).reshape(D, H * D)
    w2 = jnp.transpose(aW_w[:, D:2 * D, :], (1, 0, 2)).reshape(D, H * D)
    w3 = jnp.transpose(aW_w[:, 2 * D:, :], (1, 0, 2)).reshape(DE, H * D)
    bv = aW_b.reshape(1, H * D)
    eye16 = jax.nn.one_hot(jnp.arange(H), 16, dtype=_F32)
    ablk = (aA_w[:, :, 0][:, :, None] * eye16[:, None, :]).reshape(H * D, 16)
    ab8 = jnp.concatenate(
        [aA_b[:, 0], jnp.full((16 - H,), -1e30, _F32)]).reshape(1, 16)

    u1 = nn1_w[:D]
    u2 = nn1_w[D:2 * D]
    u3 = nn1_w[2 * D:]

    tbl = lax.bitcast_convert_type(
        bb_nodes.astype(jnp.bfloat16).reshape(N, 64, 2), jnp.int32)
    src32, snk32 = _gather_rows_kernel()(tbl, seg0, seg1)
    wcat = jnp.concatenate([w1[0::2], w1[1::2], w2[0::2], w2[1::2]], axis=0)
    ucat = jnp.concatenate([u1[0::2], u1[1::2], u2[0::2], u2[1::2]], axis=0)
    expatt, nnu = _edge_fwd(
        src32, snk32, eattr, wcat, w3, bv, ablk, ab8, ucat, u3,
        nn1_b.reshape(1, D), nn2_w, nn2_b.reshape(1, D),
        nn3_w, nn3_b.reshape(1, D))

    zeros8 = jnp.zeros((N, 16), _F32)
    an_parts = _anorm_scatter_kernel()(expatt, seg1, zeros8)
    anorm = an_parts[0] + an_parts[1]

    zeros128 = jnp.zeros((N, D), _F32)
    s_parts, ang = _agg_scatter_kernel()(nnu, seg1, segn, anorm, zeros128)

    x, ts_tab, tk_tab = _node_update(
        bb_nodes, s_parts[0], s_parts[1], expatt[:N], ang,
        dn1_w, dn1_b.reshape(1, 4 * D), dn2_w, dn2_b.reshape(1, D),
        ln1_g.reshape(1, D), ln1_b.reshape(1, D),
        eu1_w[:D], eu1_w[D:2 * D], eu1_b.reshape(1, DE))

    ts_rows, tk_rows = _gather_tables_kernel()(ts_tab, tk_tab, seg0, seg1)
    ea = _edge_out(ts_rows, tk_rows, eattr, eu1_w[2 * D:], eu2_w,
                   eu2_b.reshape(1, DE), eu3_w, eu3_b.reshape(1, DE),
                   en_g.reshape(1, DE), en_b.reshape(1, DE))
    return (x, ea)
